# baseline (speedup 1.0000x reference)
import functools

import jax
import jax.numpy as jnp
from jax import lax
from jax.experimental import pallas as pl
from jax.experimental.pallas import tpu as pltpu
from jax.experimental.pallas import tpu_sc as plsc

E = 8
K = 2
SG = 1024
D = 1024
F = 4096
G = 8
C = 256
EC = E * C
NSLOT = E * G * C
NTOK = G * SG

NC = 2
NS = 16
NW = NC * NS


def _routing_body(x_ref, wr_ref, gidx_ref, sw_ref, cidx_ref, aux_ref):
    g = pl.program_id(0)
    x = x_ref[0]
    logits = jnp.dot(x, wr_ref[...], preferred_element_type=jnp.float32)
    z = logits - jnp.max(logits, axis=1, keepdims=True)
    ez = jnp.exp(z)
    gates = ez / jnp.sum(ez, axis=1, keepdims=True)

    iota_e = lax.broadcasted_iota(jnp.int32, (SG, E), 1)
    m1 = jnp.max(gates, axis=1, keepdims=True)
    e1 = jnp.min(jnp.where(gates == m1, iota_e, E), axis=1, keepdims=True)
    g2 = jnp.where(iota_e == e1, -1.0, gates)
    m2 = jnp.max(g2, axis=1, keepdims=True)
    e2 = jnp.min(jnp.where(g2 == m2, iota_e, E), axis=1, keepdims=True)

    iota_e2 = lax.broadcasted_iota(jnp.int32, (K * SG, E), 1)
    e_t = jnp.concatenate([e1, e2], axis=0)
    oh = (iota_e2 == e_t).astype(jnp.float32)
    pos = oh
    sh = 1
    while sh < K * SG:
        pos = pos + jnp.concatenate(
            [jnp.zeros((sh, E), jnp.float32), pos[: K * SG - sh]], axis=0)
        sh *= 2
    counts = pos[K * SG - 1 : K * SG]
    pos_ex = pos - oh
    pos_t = jnp.sum(pos_ex * oh, axis=1, keepdims=True)
    v_t = pos_t < float(C)
    c_t = jnp.minimum(pos_t, float(C - 1)).astype(jnp.int32)

    j_of_t = e_t * C + c_t
    jcol = lax.broadcasted_iota(jnp.int32, (K * SG, EC), 1)
    m = jnp.where((j_of_t == jcol) & v_t, 1.0, 0.0)
    t_col = lax.broadcasted_iota(jnp.int32, (K * SG, 1), 0)
    s_col = (t_col - jnp.where(t_col >= SG, SG, 0) + SG * g).astype(jnp.float32)
    w_col = jnp.concatenate([m1, m2], axis=0)
    gidx_row = jnp.sum(s_col * m, axis=0, keepdims=True)
    sw_row = jnp.sum(w_col * m, axis=0, keepdims=True)
    gidx_ref[0] = gidx_row.astype(jnp.int32)
    sw_ref[0] = sw_row

    cmin = jnp.min(counts, axis=1, keepdims=True)
    lane8 = lax.broadcasted_iota(jnp.int32, (1, E), 1)
    estar = jnp.min(jnp.where(counts == cmin, lane8, E), axis=1, keepdims=True)
    redirect = estar * (G * C) + g * C + (C - 1)
    j0 = e1 * (G * C) + g * C + c_t[:SG]
    j1 = e2 * (G * C) + g * C + c_t[SG:]
    c0 = jnp.where(v_t[:SG], j0, redirect)
    c1 = jnp.where(v_t[SG:], j1, redirect)
    cidx_ref[0] = jnp.concatenate([c0, c1], axis=1)

    imp = jnp.sum(gates, axis=0, keepdims=True)
    mean = jnp.mean(imp, axis=1, keepdims=True)
    var = jnp.mean((imp - mean) ** 2, axis=1, keepdims=True)
    il = var / ((mean + 1e-10) ** 2)

    @pl.when(g == 0)
    def _():
        aux_ref[...] = jnp.zeros_like(aux_ref)

    aux_ref[...] += il / float(G)


def _routing(x3, w_router):
    return pl.pallas_call(
        _routing_body,
        grid=(G,),
        in_specs=[
            pl.BlockSpec((1, SG, D), lambda g: (g, 0, 0)),
            pl.BlockSpec((D, E), lambda g: (0, 0)),
        ],
        out_specs=[
            pl.BlockSpec((1, 1, EC), lambda g: (g, 0, 0)),
            pl.BlockSpec((1, 1, EC), lambda g: (g, 0, 0)),
            pl.BlockSpec((1, SG, K), lambda g: (g, 0, 0)),
            pl.BlockSpec((1, 1), lambda g: (0, 0)),
        ],
        out_shape=[
            jax.ShapeDtypeStruct((G, 1, EC), jnp.int32),
            jax.ShapeDtypeStruct((G, 1, EC), jnp.float32),
            jax.ShapeDtypeStruct((G, SG, K), jnp.int32),
            jax.ShapeDtypeStruct((1, 1), jnp.float32),
        ],
    )(x3, w_router)


_G_RPW = NSLOT // NW
_G_CH = 64


@functools.partial(
    pl.kernel,
    mesh=plsc.VectorSubcoreMesh(core_axis_name="c", subcore_axis_name="s"),
    out_type=jax.ShapeDtypeStruct((NSLOT, D), jnp.float32),
    scratch_types=[
        pltpu.VMEM((_G_CH,), jnp.int32),
        pltpu.VMEM((_G_CH, D), jnp.float32),
        pltpu.SemaphoreType.DMA,
    ],
)
def _sc_gather(x_hbm, idx_hbm, out_hbm, idx_v, rows_v, sem):
    wid = lax.axis_index("s") * NC + lax.axis_index("c")
    base = wid * _G_RPW

    def chunk(i, carry):
        off = base + i * _G_CH
        pltpu.sync_copy(idx_hbm.at[pl.ds(off, _G_CH)], idx_v)
        pltpu.async_copy(x_hbm.at[idx_v], rows_v, sem).wait()
        pltpu.sync_copy(rows_v, out_hbm.at[pl.ds(off, _G_CH)])
        return carry

    lax.fori_loop(0, _G_RPW // _G_CH, chunk, 0)


_RT = 512
_FT = 2048
_NR = (G * C) // _RT
_NF = F // _FT


def _mlp_body(x_ref, w1_ref, b1_ref, w2_ref, b2_ref, sw_ref, out_ref):
    f = pl.program_id(2)
    h = jnp.dot(x_ref[0].astype(jnp.bfloat16), w1_ref[0].astype(jnp.bfloat16),
                preferred_element_type=jnp.float32)
    h = jax.nn.gelu(h + b1_ref[0])
    part = jnp.dot(h.astype(jnp.bfloat16), w2_ref[0].astype(jnp.bfloat16),
                   preferred_element_type=jnp.float32)

    @pl.when(f == 0)
    def _():
        out_ref[0] = part

    @pl.when(f > 0)
    def _():
        out_ref[0] += part

    @pl.when(f == _NF - 1)
    def _():
        out_ref[0] = (out_ref[0] + b2_ref[0]) * sw_ref[0]


def _mlp(xe, w1, b1, w2, b2, sw_col):
    return pl.pallas_call(
        _mlp_body,
        grid=(E, _NR, _NF),
        in_specs=[
            pl.BlockSpec((1, _RT, D), lambda e, r, f: (e, r, 0)),
            pl.BlockSpec((1, D, _FT), lambda e, r, f: (e, 0, f)),
            pl.BlockSpec((1, 1, _FT), lambda e, r, f: (e, 0, f)),
            pl.BlockSpec((1, _FT, D), lambda e, r, f: (e, f, 0)),
            pl.BlockSpec((1, 1, D), lambda e, r, f: (e, 0, 0)),
            pl.BlockSpec((1, _RT, 1), lambda e, r, f: (e, r, 0)),
        ],
        out_specs=pl.BlockSpec((1, _RT, D), lambda e, r, f: (e, r, 0)),
        out_shape=jax.ShapeDtypeStruct((E, G * C, D), jnp.float32),
    )(xe, w1, b1, w2, b2, sw_col)


_C_TPW = NTOK // NW
_C_CH = 32


@functools.partial(
    pl.kernel,
    mesh=plsc.VectorSubcoreMesh(core_axis_name="c", subcore_axis_name="s"),
    out_type=jax.ShapeDtypeStruct((NTOK, D), jnp.float32),
    scratch_types=[
        pltpu.VMEM((_C_CH,), jnp.int32),
        pltpu.VMEM((_C_CH,), jnp.int32),
        pltpu.VMEM((_C_CH, D), jnp.float32),
        pltpu.VMEM((_C_CH, D), jnp.float32),
        pltpu.SemaphoreType.DMA,
        pltpu.SemaphoreType.DMA,
    ],
)
def _sc_combine(eo_hbm, i0_hbm, i1_hbm, out_hbm, i0_v, i1_v, r0_v, r1_v, s0, s1):
    wid = lax.axis_index("s") * NC + lax.axis_index("c")
    base = wid * _C_TPW

    def chunk(i, carry):
        off = base + i * _C_CH
        pltpu.sync_copy(i0_hbm.at[pl.ds(off, _C_CH)], i0_v)
        pltpu.sync_copy(i1_hbm.at[pl.ds(off, _C_CH)], i1_v)
        cp0 = pltpu.async_copy(eo_hbm.at[i0_v], r0_v, s0)
        cp1 = pltpu.async_copy(eo_hbm.at[i1_v], r1_v, s1)
        cp0.wait()
        cp1.wait()

        def add16(j, carry2):
            r = j // (D // 16)
            c = (j % (D // 16)) * 16
            r0_v[r, pl.ds(c, 16)] = r0_v[r, pl.ds(c, 16)] + r1_v[r, pl.ds(c, 16)]
            return carry2

        lax.fori_loop(0, _C_CH * (D // 16), add16, 0)
        pltpu.sync_copy(r0_v, out_hbm.at[pl.ds(off, _C_CH)])
        return carry

    lax.fori_loop(0, _C_TPW // _C_CH, chunk, 0)


def kernel(inputs, w_router, w1, b1, w2, b2):
    B, S, _ = inputs.shape
    x3 = inputs.reshape(G, SG, D)

    gidx, sw, cidx, aux = _routing(x3, w_router)

    gidx_flat = gidx.reshape(G, E, C).transpose(1, 0, 2).reshape(NSLOT)
    sw_col = sw.reshape(G, E, C).transpose(1, 0, 2).reshape(E, G * C, 1)
    cidx_t = cidx.transpose(2, 0, 1).reshape(K, NTOK)

    xe_flat = _sc_gather(x3.reshape(NTOK, D), gidx_flat)
    eo = _mlp(xe_flat.reshape(E, G * C, D), w1, b1.reshape(E, 1, F), w2,
              b2.reshape(E, 1, D), sw_col)
    out = _sc_combine(eo.reshape(NSLOT, D), cidx_t[0], cidx_t[1])

    return out.reshape(B, S, D), aux.reshape(())


# speedup vs baseline: 1.1113x; 1.1113x over previous
import functools

import jax
import jax.numpy as jnp
from jax import lax
from jax.experimental import pallas as pl
from jax.experimental.pallas import tpu as pltpu
from jax.experimental.pallas import tpu_sc as plsc

E = 8
K = 2
SG = 1024
D = 1024
F = 4096
G = 8
C = 256
EC = E * C
NSLOT = E * G * C
NTOK = G * SG

NC = 2
NS = 16
NW = NC * NS


def _routing_body(x_ref, wr_ref, gidx_ref, sw_ref, cidx_ref, aux_ref):
    g = pl.program_id(0)
    x = x_ref[0]
    logits = jnp.dot(x, wr_ref[...], preferred_element_type=jnp.float32)
    z = logits - jnp.max(logits, axis=1, keepdims=True)
    ez = jnp.exp(z)
    gates = ez / jnp.sum(ez, axis=1, keepdims=True)

    iota_e = lax.broadcasted_iota(jnp.int32, (SG, E), 1)
    m1 = jnp.max(gates, axis=1, keepdims=True)
    e1 = jnp.min(jnp.where(gates == m1, iota_e, E), axis=1, keepdims=True)
    g2 = jnp.where(iota_e == e1, -1.0, gates)
    m2 = jnp.max(g2, axis=1, keepdims=True)
    e2 = jnp.min(jnp.where(g2 == m2, iota_e, E), axis=1, keepdims=True)

    iota_e2 = lax.broadcasted_iota(jnp.int32, (K * SG, E), 1)
    e_t = jnp.concatenate([e1, e2], axis=0)
    oh = (iota_e2 == e_t).astype(jnp.float32)
    pos = oh
    sh = 1
    while sh < K * SG:
        pos = pos + jnp.concatenate(
            [jnp.zeros((sh, E), jnp.float32), pos[: K * SG - sh]], axis=0)
        sh *= 2
    counts = pos[K * SG - 1 : K * SG]
    pos_ex = pos - oh
    pos_t = jnp.sum(pos_ex * oh, axis=1, keepdims=True)
    v_t = pos_t < float(C)
    c_t = jnp.minimum(pos_t, float(C - 1)).astype(jnp.int32)

    j_of_t = e_t * C + c_t
    jcol = lax.broadcasted_iota(jnp.int32, (K * SG, EC), 1)
    m = jnp.where((j_of_t == jcol) & v_t, 1.0, 0.0)
    t_col = lax.broadcasted_iota(jnp.int32, (K * SG, 1), 0)
    s_col = (t_col - jnp.where(t_col >= SG, SG, 0) + SG * g).astype(jnp.float32)
    w_col = jnp.concatenate([m1, m2], axis=0)
    gidx_row = jnp.sum(s_col * m, axis=0, keepdims=True)
    sw_row = jnp.sum(w_col * m, axis=0, keepdims=True)
    gidx_ref[0] = gidx_row.astype(jnp.int32)
    sw_ref[0] = sw_row

    cmin = jnp.min(counts, axis=1, keepdims=True)
    lane8 = lax.broadcasted_iota(jnp.int32, (1, E), 1)
    estar = jnp.min(jnp.where(counts == cmin, lane8, E), axis=1, keepdims=True)
    redirect = estar * (G * C) + g * C + (C - 1)
    j0 = e1 * (G * C) + g * C + c_t[:SG]
    j1 = e2 * (G * C) + g * C + c_t[SG:]
    c0 = jnp.where(v_t[:SG], j0, redirect)
    c1 = jnp.where(v_t[SG:], j1, redirect)
    cidx_ref[0] = jnp.concatenate([c0, c1], axis=1)

    imp = jnp.sum(gates, axis=0, keepdims=True)
    mean = jnp.mean(imp, axis=1, keepdims=True)
    var = jnp.mean((imp - mean) ** 2, axis=1, keepdims=True)
    il = var / ((mean + 1e-10) ** 2)

    @pl.when(g == 0)
    def _():
        aux_ref[...] = jnp.zeros_like(aux_ref)

    aux_ref[...] += il / float(G)


def _routing(x3, w_router):
    return pl.pallas_call(
        _routing_body,
        grid=(G,),
        in_specs=[
            pl.BlockSpec((1, SG, D), lambda g: (g, 0, 0)),
            pl.BlockSpec((D, E), lambda g: (0, 0)),
        ],
        out_specs=[
            pl.BlockSpec((1, 1, EC), lambda g: (g, 0, 0)),
            pl.BlockSpec((1, 1, EC), lambda g: (g, 0, 0)),
            pl.BlockSpec((1, SG, K), lambda g: (g, 0, 0)),
            pl.BlockSpec((1, 1), lambda g: (0, 0)),
        ],
        out_shape=[
            jax.ShapeDtypeStruct((G, 1, EC), jnp.int32),
            jax.ShapeDtypeStruct((G, 1, EC), jnp.float32),
            jax.ShapeDtypeStruct((G, SG, K), jnp.int32),
            jax.ShapeDtypeStruct((1, 1), jnp.float32),
        ],
    )(x3, w_router)


_G_RPW = NSLOT // NW
_G_CH = 32
_G_NCH = _G_RPW // _G_CH


@functools.partial(
    pl.kernel,
    mesh=plsc.VectorSubcoreMesh(core_axis_name="c", subcore_axis_name="s"),
    out_type=jax.ShapeDtypeStruct((NSLOT, D), jnp.float32),
    scratch_types=[
        pltpu.VMEM((_G_RPW,), jnp.int32),
        pltpu.VMEM((_G_CH, D), jnp.float32),
        pltpu.VMEM((_G_CH, D), jnp.float32),
        pltpu.SemaphoreType.DMA,
        pltpu.SemaphoreType.DMA,
        pltpu.SemaphoreType.DMA,
    ],
)
def _sc_gather(x_hbm, idx_hbm, out_hbm, idx_v, rows0_v, rows1_v, gsem, ws0, ws1):
    wid = lax.axis_index("s") * NC + lax.axis_index("c")
    base = wid * _G_RPW
    pltpu.sync_copy(idx_hbm.at[pl.ds(base, _G_RPW)], idx_v)
    rows = (rows0_v, rows1_v)
    wsem = (ws0, ws1)
    pending = [None, None]
    for i in range(_G_NCH):
        b = i % 2
        if pending[b] is not None:
            pending[b].wait()
        pltpu.async_copy(
            x_hbm.at[idx_v.at[pl.ds(i * _G_CH, _G_CH)]], rows[b], gsem).wait()
        pending[b] = pltpu.async_copy(
            rows[b], out_hbm.at[pl.ds(base + i * _G_CH, _G_CH)], wsem[b])
    pending[0].wait()
    pending[1].wait()


_RT = 512
_FT = 2048
_NR = (G * C) // _RT
_NF = F // _FT


def _mlp_body(x_ref, w1_ref, b1_ref, w2_ref, b2_ref, sw_ref, out_ref):
    f = pl.program_id(2)
    h = jnp.dot(x_ref[0].astype(jnp.bfloat16), w1_ref[0].astype(jnp.bfloat16),
                preferred_element_type=jnp.float32)
    h = jax.nn.gelu((h + b1_ref[0]).astype(jnp.bfloat16))
    part = jnp.dot(h, w2_ref[0].astype(jnp.bfloat16),
                   preferred_element_type=jnp.float32)

    @pl.when(f == 0)
    def _():
        out_ref[0] = part

    @pl.when(f > 0)
    def _():
        out_ref[0] += part

    @pl.when(f == _NF - 1)
    def _():
        out_ref[0] = (out_ref[0] + b2_ref[0]) * sw_ref[0]


def _mlp(xe, w1, b1, w2, b2, sw_col):
    return pl.pallas_call(
        _mlp_body,
        grid=(E, _NR, _NF),
        in_specs=[
            pl.BlockSpec((1, _RT, D), lambda e, r, f: (e, r, 0)),
            pl.BlockSpec((1, D, _FT), lambda e, r, f: (e, 0, f)),
            pl.BlockSpec((1, 1, _FT), lambda e, r, f: (e, 0, f)),
            pl.BlockSpec((1, _FT, D), lambda e, r, f: (e, f, 0)),
            pl.BlockSpec((1, 1, D), lambda e, r, f: (e, 0, 0)),
            pl.BlockSpec((1, _RT, 1), lambda e, r, f: (e, r, 0)),
        ],
        out_specs=pl.BlockSpec((1, _RT, D), lambda e, r, f: (e, r, 0)),
        out_shape=jax.ShapeDtypeStruct((E, G * C, D), jnp.float32),
    )(xe, w1, b1, w2, b2, sw_col)


_C_TPW = NTOK // NW
_C_CH = 16
_C_NCH = _C_TPW // _C_CH


@functools.partial(
    pl.kernel,
    mesh=plsc.VectorSubcoreMesh(core_axis_name="c", subcore_axis_name="s"),
    out_type=jax.ShapeDtypeStruct((NTOK, D), jnp.float32),
    scratch_types=[
        pltpu.VMEM((_C_TPW,), jnp.int32),
        pltpu.VMEM((_C_TPW,), jnp.int32),
        pltpu.VMEM((_C_CH, D), jnp.float32),
        pltpu.VMEM((_C_CH, D), jnp.float32),
        pltpu.VMEM((_C_CH, D), jnp.float32),
        pltpu.VMEM((_C_CH, D), jnp.float32),
        pltpu.SemaphoreType.DMA,
        pltpu.SemaphoreType.DMA,
        pltpu.SemaphoreType.DMA,
        pltpu.SemaphoreType.DMA,
    ],
)
def _sc_combine(eo_hbm, i0_hbm, i1_hbm, out_hbm, i0_v, i1_v,
                r0a_v, r1a_v, r0b_v, r1b_v, g0, g1, wsa, wsb):
    wid = lax.axis_index("s") * NC + lax.axis_index("c")
    base = wid * _C_TPW
    pltpu.sync_copy(i0_hbm.at[pl.ds(base, _C_TPW)], i0_v)
    pltpu.sync_copy(i1_hbm.at[pl.ds(base, _C_TPW)], i1_v)
    r0 = (r0a_v, r0b_v)
    r1 = (r1a_v, r1b_v)
    wsem = (wsa, wsb)
    pending = [None, None]
    for i in range(_C_NCH):
        b = i % 2
        if pending[b] is not None:
            pending[b].wait()
        cp0 = pltpu.async_copy(eo_hbm.at[i0_v.at[pl.ds(i * _C_CH, _C_CH)]], r0[b], g0)
        cp1 = pltpu.async_copy(eo_hbm.at[i1_v.at[pl.ds(i * _C_CH, _C_CH)]], r1[b], g1)
        cp0.wait()
        cp1.wait()
        a, bb = r0[b], r1[b]

        def addrow(r, carry, a=a, bb=bb):
            for u in range(D // 16):
                c = u * 16
                a[r, pl.ds(c, 16)] = a[r, pl.ds(c, 16)] + bb[r, pl.ds(c, 16)]
            return carry

        lax.fori_loop(0, _C_CH, addrow, 0)
        pending[b] = pltpu.async_copy(
            r0[b], out_hbm.at[pl.ds(base + i * _C_CH, _C_CH)], wsem[b])
    pending[0].wait()
    pending[1].wait()


def kernel(inputs, w_router, w1, b1, w2, b2):
    B, S, _ = inputs.shape
    x3 = inputs.reshape(G, SG, D)

    gidx, sw, cidx, aux = _routing(x3, w_router)

    gidx_flat = gidx.reshape(G, E, C).transpose(1, 0, 2).reshape(NSLOT)
    sw_col = sw.reshape(G, E, C).transpose(1, 0, 2).reshape(E, G * C, 1)
    cidx_t = cidx.transpose(2, 0, 1).reshape(K, NTOK)

    xe_flat = _sc_gather(x3.reshape(NTOK, D), gidx_flat)
    eo = _mlp(xe_flat.reshape(E, G * C, D), w1, b1.reshape(E, 1, F), w2,
              b2.reshape(E, 1, D), sw_col)
    out = _sc_combine(eo.reshape(NSLOT, D), cidx_t[0], cidx_t[1])

    return out.reshape(B, S, D), aux.reshape(())
